# baseline (speedup 1.0000x reference)
import functools

import jax
import jax.numpy as jnp
from jax import lax
from jax.experimental import pallas as pl
from jax.experimental.pallas import tpu as pltpu
from jax.experimental.pallas import tpu_sc as plsc

N_NODES_C = 100000
N_EDGES_C = 6400000
_EPS = 1.0
_SIG = 1.0
_CUT = 5.0
_HALF_SHIFT = 2.0 * _EPS * ((_SIG / _CUT) ** 12 - (_SIG / _CUT) ** 6)

NC = 2
NS = 16
NW = NC * NS
NHALF = 2
EDGES_H = N_EDGES_C // NHALF
EPW = EDGES_H // NW
CHUNK = 2000
NCHUNK = EPW // CHUNK
NPAIR = NCHUNK // 2
GROUPS = CHUNK // 16
UNROLL = 5

ACC = 100352


def _sc_body(phase, x_hbm, y_hbm, z_hbm, i_hbm, j_hbm, out_hbm,
             acc, x0, y0, z0, i0, j0, x1, y1, z1, i1, j1, sem0, sem1):
    cid = lax.axis_index("c")
    sid = lax.axis_index("s")
    wid = cid * NS + sid
    pbase = wid * EPW
    ebase = phase * EDGES_H + wid * EPW

    slots = ((x0, y0, z0, i0, j0, sem0), (x1, y1, z1, i1, j1, sem1))
    hbms = (x_hbm, y_hbm, z_hbm, i_hbm, j_hbm)

    def issue5(slot, k):
        for n, (hbm, buf) in enumerate(zip(hbms, slot[:5])):
            base = (pbase if n < 3 else ebase) + k * CHUNK
            pltpu.async_copy(hbm.at[pl.ds(base, CHUNK)], buf, slot[5])

    def wait5(slot):
        for hbm, buf in zip(hbms, slot[:5]):
            pltpu.make_async_copy(hbm.at[pl.ds(0, CHUNK)], buf, slot[5]).wait()

    issue5(slots[0], 0)

    zero16 = jnp.zeros((16,), jnp.float32)

    @plsc.parallel_loop(0, ACC // 64, 1, unroll=4)
    def _zero(t):
        for q in range(4):
            acc[pl.ds(t * 64 + q * 16, 16)] = zero16

    def compute(slot):
        xb, yb, zb, ib, jb = slot[:5]

        @plsc.parallel_loop(0, GROUPS, 1, unroll=UNROLL)
        def _group(g):
            o = g * 16
            dx = xb[pl.ds(o, 16)]
            dy = yb[pl.ds(o, 16)]
            dz = zb[pl.ds(o, 16)]
            r2 = dx * dx + dy * dy + dz * dz
            inv = 1.0 / r2
            s6 = inv * inv * inv
            he = 2.0 * _EPS * (s6 * s6 - s6) - _HALF_SHIFT
            iv = ib[pl.ds(o, 16)]
            jv = jb[pl.ds(o, 16)]
            plsc.addupdate_scatter(acc, [iv], he)
            plsc.addupdate_scatter(acc, [jv], he)

    def _pair(t, _):
        issue5(slots[1], 2 * t + 1)
        wait5(slots[0])
        compute(slots[0])

        @pl.when(t < NPAIR - 1)
        def _():
            issue5(slots[0], 2 * t + 2)

        wait5(slots[1])
        compute(slots[1])
        return 0

    lax.fori_loop(0, NPAIR, _pair, 0)

    pltpu.sync_copy(acc, out_hbm.at[wid])


def _sc_partials(phase, xs, ys, zs, all_i, all_j):
    mesh = plsc.VectorSubcoreMesh(core_axis_name="c", subcore_axis_name="s",
                                  num_cores=NC, num_subcores=NS)
    return pl.kernel(
        functools.partial(_sc_body, phase),
        out_type=jax.ShapeDtypeStruct((NW, ACC), jnp.float32),
        mesh=mesh,
        compiler_params=pltpu.CompilerParams(needs_layout_passes=False,
                                             use_tc_tiling_on_sc=False),
        scratch_types=[
            pltpu.VMEM((ACC,), jnp.float32),
            pltpu.VMEM((CHUNK,), jnp.float32),
            pltpu.VMEM((CHUNK,), jnp.float32),
            pltpu.VMEM((CHUNK,), jnp.float32),
            pltpu.VMEM((CHUNK,), jnp.int32),
            pltpu.VMEM((CHUNK,), jnp.int32),
            pltpu.VMEM((CHUNK,), jnp.float32),
            pltpu.VMEM((CHUNK,), jnp.float32),
            pltpu.VMEM((CHUNK,), jnp.float32),
            pltpu.VMEM((CHUNK,), jnp.int32),
            pltpu.VMEM((CHUNK,), jnp.int32),
            pltpu.SemaphoreType.DMA,
            pltpu.SemaphoreType.DMA,
        ],
    )(xs, ys, zs, all_i, all_j)


def _sum_body(p0_ref, p1_ref, b_ref, o_ref):
    o_ref[...] = (jnp.sum(p0_ref[...], axis=0) + jnp.sum(p1_ref[...], axis=0)
                  + b_ref[...])


def _tc_sum(p0, p1, bias):
    return pl.pallas_call(
        _sum_body,
        out_shape=jax.ShapeDtypeStruct((ACC,), jnp.float32),
    )(p0, p1, bias)


def kernel(distances, all_i, all_j, n_nodes):
    H = EDGES_H
    parts = []
    for ph in range(NHALF):
        xs = distances[ph * H:(ph + 1) * H, 0]
        ys = distances[ph * H:(ph + 1) * H, 1]
        zs = distances[ph * H:(ph + 1) * H, 2]
        parts.append(_sc_partials(ph, xs, ys, zs, all_i, all_j))
    bias = jnp.full((1,), 0.0, jnp.float32) + (
        jnp.asarray(n_nodes, jnp.float32) - float(N_NODES_C))
    summed = _tc_sum(parts[0], parts[1], bias)
    return summed[:N_NODES_C].reshape(-1, 1)


# speedup vs baseline: 1.0880x; 1.0880x over previous
import functools

import jax
import jax.numpy as jnp
from jax import lax
from jax.experimental import pallas as pl
from jax.experimental.pallas import tpu as pltpu
from jax.experimental.pallas import tpu_sc as plsc

N_NODES_C = 100000
N_EDGES_C = 6400000
_EPS = 1.0
_SIG = 1.0
_CUT = 5.0
_HALF_SHIFT = 2.0 * _EPS * ((_SIG / _CUT) ** 12 - (_SIG / _CUT) ** 6)

NC = 2
NS = 16
NW = NC * NS
NHALF = 1
EDGES_H = N_EDGES_C // NHALF
EPW = EDGES_H // NW
CHUNK = 2000
NCHUNK = EPW // CHUNK
NPAIR = NCHUNK // 2
GROUPS = CHUNK // 16
UNROLL = 5

ACC = 100352


def _sc_body(phase, x_hbm, y_hbm, z_hbm, i_hbm, j_hbm, out_hbm,
             acc, x0, y0, z0, i0, j0, x1, y1, z1, i1, j1, sem0, sem1):
    cid = lax.axis_index("c")
    sid = lax.axis_index("s")
    wid = cid * NS + sid
    pbase = wid * EPW
    ebase = phase * EDGES_H + wid * EPW

    slots = ((x0, y0, z0, i0, j0, sem0), (x1, y1, z1, i1, j1, sem1))
    hbms = (x_hbm, y_hbm, z_hbm, i_hbm, j_hbm)

    def issue5(slot, k):
        for n, (hbm, buf) in enumerate(zip(hbms, slot[:5])):
            base = (pbase if n < 3 else ebase) + k * CHUNK
            pltpu.async_copy(hbm.at[pl.ds(base, CHUNK)], buf, slot[5])

    def wait5(slot):
        for hbm, buf in zip(hbms, slot[:5]):
            pltpu.make_async_copy(hbm.at[pl.ds(0, CHUNK)], buf, slot[5]).wait()

    issue5(slots[0], 0)

    zero16 = jnp.zeros((16,), jnp.float32)

    @plsc.parallel_loop(0, ACC // 64, 1, unroll=4)
    def _zero(t):
        for q in range(4):
            acc[pl.ds(t * 64 + q * 16, 16)] = zero16

    def compute(slot):
        xb, yb, zb, ib, jb = slot[:5]

        @plsc.parallel_loop(0, GROUPS, 1, unroll=UNROLL)
        def _group(g):
            o = g * 16
            dx = xb[pl.ds(o, 16)]
            dy = yb[pl.ds(o, 16)]
            dz = zb[pl.ds(o, 16)]
            r2 = dx * dx + dy * dy + dz * dz
            inv = 1.0 / r2
            s6 = inv * inv * inv
            he = 2.0 * _EPS * (s6 * s6 - s6) - _HALF_SHIFT
            iv = ib[pl.ds(o, 16)]
            jv = jb[pl.ds(o, 16)]
            plsc.addupdate_scatter(acc, [iv], he)
            plsc.addupdate_scatter(acc, [jv], he)

    def _pair(t, _):
        issue5(slots[1], 2 * t + 1)
        wait5(slots[0])
        compute(slots[0])

        @pl.when(t < NPAIR - 1)
        def _():
            issue5(slots[0], 2 * t + 2)

        wait5(slots[1])
        compute(slots[1])
        return 0

    lax.fori_loop(0, NPAIR, _pair, 0)

    pltpu.sync_copy(acc, out_hbm.at[wid])


def _sc_partials(phase, xs, ys, zs, all_i, all_j):
    mesh = plsc.VectorSubcoreMesh(core_axis_name="c", subcore_axis_name="s",
                                  num_cores=NC, num_subcores=NS)
    return pl.kernel(
        functools.partial(_sc_body, phase),
        out_type=jax.ShapeDtypeStruct((NW, ACC), jnp.float32),
        mesh=mesh,
        compiler_params=pltpu.CompilerParams(needs_layout_passes=False,
                                             use_tc_tiling_on_sc=False),
        scratch_types=[
            pltpu.VMEM((ACC,), jnp.float32),
            pltpu.VMEM((CHUNK,), jnp.float32),
            pltpu.VMEM((CHUNK,), jnp.float32),
            pltpu.VMEM((CHUNK,), jnp.float32),
            pltpu.VMEM((CHUNK,), jnp.int32),
            pltpu.VMEM((CHUNK,), jnp.int32),
            pltpu.VMEM((CHUNK,), jnp.float32),
            pltpu.VMEM((CHUNK,), jnp.float32),
            pltpu.VMEM((CHUNK,), jnp.float32),
            pltpu.VMEM((CHUNK,), jnp.int32),
            pltpu.VMEM((CHUNK,), jnp.int32),
            pltpu.SemaphoreType.DMA,
            pltpu.SemaphoreType.DMA,
        ],
    )(xs, ys, zs, all_i, all_j)


def _sum_body(*refs):
    *p_refs, b_ref, o_ref = refs
    s = b_ref[...]
    for p_ref in p_refs:
        s = s + jnp.sum(p_ref[...], axis=0)
    o_ref[...] = s


def _tc_sum(parts, bias):
    return pl.pallas_call(
        _sum_body,
        out_shape=jax.ShapeDtypeStruct((ACC,), jnp.float32),
    )(*parts, bias)


def kernel(distances, all_i, all_j, n_nodes):
    H = EDGES_H
    parts = []
    for ph in range(NHALF):
        xs = distances[ph * H:(ph + 1) * H, 0]
        ys = distances[ph * H:(ph + 1) * H, 1]
        zs = distances[ph * H:(ph + 1) * H, 2]
        parts.append(_sc_partials(ph, xs, ys, zs, all_i, all_j))
    bias = jnp.full((1,), 0.0, jnp.float32) + (
        jnp.asarray(n_nodes, jnp.float32) - float(N_NODES_C))
    summed = _tc_sum(parts, bias)
    return summed[:N_NODES_C].reshape(-1, 1)
